# baseline (speedup 1.0000x reference)
import jax
import jax.numpy as jnp
from jax.experimental import pallas as pl
from jax.experimental.pallas import tpu as pltpu

_E, _B, _S, _C = 10, 128, 4000, 5
_SB = 400
_CH = 40


def _vote_kernel(x_ref, n_ref, o_ref):
    for rr in range(_SB // _CH):
        r0 = rr * _CH
        acc = [None] * _C
        for e in range(_E):
            xc = [x_ref[e, c, r0:r0 + _CH, :] for c in range(_C)]
            w = 1.0 + n_ref[e, r0:r0 + _CH, :] * 0.001
            p2 = jnp.maximum(xc[0], xc[1])
            p3 = jnp.maximum(p2, xc[2])
            p4 = jnp.maximum(p3, xc[3])
            m = jnp.maximum(p4, xc[4])
            v = [xc[0] == m,
                 (xc[1] == m) & (xc[0] < m),
                 (xc[2] == m) & (p2 < m),
                 (xc[3] == m) & (p3 < m),
                 p4 < m]
            for c in range(_C):
                contrib = jnp.where(v[c], w, 0.0)
                acc[c] = contrib if acc[c] is None else acc[c] + contrib
        for c in range(_C):
            o_ref[c, r0:r0 + _CH, :] = acc[c]


def kernel(expert_logits, noise):
    E, B, S, C = expert_logits.shape
    xt = jnp.transpose(expert_logits, (0, 3, 2, 1))
    nt = jnp.transpose(noise, (0, 2, 1))
    out = pl.pallas_call(
        _vote_kernel,
        grid=(_S // _SB,),
        in_specs=[
            pl.BlockSpec((E, C, _SB, B), lambda i: (0, 0, i, 0)),
            pl.BlockSpec((E, _SB, B), lambda i: (0, i, 0)),
        ],
        out_specs=pl.BlockSpec((C, _SB, B), lambda i: (0, i, 0)),
        out_shape=jax.ShapeDtypeStruct((C, S, B), expert_logits.dtype),
        compiler_params=pltpu.CompilerParams(
            dimension_semantics=("arbitrary",),
        ),
    )(xt, nt)
    return jnp.transpose(out, (2, 1, 0))


# speedup vs baseline: 1.0172x; 1.0172x over previous
import jax
import jax.numpy as jnp
from jax.experimental import pallas as pl
from jax.experimental.pallas import tpu as pltpu

_E, _B, _S, _C = 10, 128, 4000, 5
_SB = 400
_CH = 8


def _vote_kernel(x_ref, n_ref, o_ref):
    for rr in range(_SB // _CH):
        r0 = rr * _CH
        acc = [None] * _C
        for e in range(_E):
            xc = [x_ref[e, c, r0:r0 + _CH, :] for c in range(_C)]
            w = 1.0 + n_ref[e, r0:r0 + _CH, :] * 0.001
            p2 = jnp.maximum(xc[0], xc[1])
            p3 = jnp.maximum(p2, xc[2])
            p4 = jnp.maximum(p3, xc[3])
            m = jnp.maximum(p4, xc[4])
            v = [xc[0] == m,
                 (xc[1] == m) & (xc[0] < m),
                 (xc[2] == m) & (p2 < m),
                 (xc[3] == m) & (p3 < m),
                 p4 < m]
            for c in range(_C):
                contrib = jnp.where(v[c], w, 0.0)
                acc[c] = contrib if acc[c] is None else acc[c] + contrib
        for c in range(_C):
            o_ref[c, r0:r0 + _CH, :] = acc[c]


def kernel(expert_logits, noise):
    E, B, S, C = expert_logits.shape
    xt = jnp.transpose(expert_logits, (0, 3, 2, 1))
    nt = jnp.transpose(noise, (0, 2, 1))
    out = pl.pallas_call(
        _vote_kernel,
        grid=(_S // _SB,),
        in_specs=[
            pl.BlockSpec((E, C, _SB, B), lambda i: (0, 0, i, 0)),
            pl.BlockSpec((E, _SB, B), lambda i: (0, i, 0)),
        ],
        out_specs=pl.BlockSpec((C, _SB, B), lambda i: (0, i, 0)),
        out_shape=jax.ShapeDtypeStruct((C, S, B), expert_logits.dtype),
        compiler_params=pltpu.CompilerParams(
            dimension_semantics=("arbitrary",),
        ),
    )(xt, nt)
    return jnp.transpose(out, (2, 1, 0))
